# baseline (speedup 1.0000x reference)
import functools

import jax
import jax.numpy as jnp
from jax import lax
from jax.experimental import pallas as pl
from jax.experimental.pallas import tpu as pltpu
from jax.experimental.pallas import tpu_sc as plsc

L = 16
D = 64
W = 128
BC = 128
NG = 4
NO = 2


def _sc_embed_norm(table2, idx_t, *, batch, hist):
    info = plsc.get_sparse_core_info()
    nc, ns = info.num_cores, info.num_subcores
    nw = nc * ns
    b_per_w = batch // nw
    kpw = b_per_w // BC
    n_chunks = hist * kpw
    assert n_chunks % NG == 0

    mesh = plsc.VectorSubcoreMesh(core_axis_name="c", subcore_axis_name="s")

    @functools.partial(
        pl.kernel,
        out_type=jax.ShapeDtypeStruct((hist, D, batch), jnp.float32),
        mesh=mesh,
        scratch_types=[
            pltpu.VMEM((hist, b_per_w), jnp.int32),
            pltpu.VMEM((NG, W), jnp.int32),
            pltpu.VMEM((NG, BC, W), jnp.float32),
            pltpu.VMEM((NO, D, BC), jnp.float32),
            pltpu.SemaphoreType.DMA((NG,)),
            pltpu.SemaphoreType.DMA((NO,)),
        ],
        compiler_params=pltpu.CompilerParams(
            needs_layout_passes=False, use_tc_tiling_on_sc=True),
    )
    def k(table_hbm, idx_hbm, out_hbm, idx_v, m_v, rows_v, obuf, gsem, ssem):
        iota = lax.iota(jnp.int32, L)
        wid = lax.axis_index("s") * nc + lax.axis_index("c")
        wb0 = wid * b_per_w
        pltpu.sync_copy(idx_hbm.at[:, pl.ds(wb0, b_per_w)], idx_v)

        def prep_gather(c, gb):
            h = c // kpw
            kk = c % kpw
            for i in range(W // L):
                m_v[gb, pl.ds(i * L, L)] = (
                    idx_v[h, pl.ds(kk * BC + i * L, L)] >> jnp.int32(1))
            pltpu.async_copy(table_hbm.at[m_v.at[gb]], rows_v.at[gb],
                             gsem.at[gb])

        def drain_gather(gb):
            pltpu.make_async_copy(table_hbm.at[m_v.at[gb]], rows_v.at[gb],
                                  gsem.at[gb]).wait()

        def store_slice(c, ob):
            h = c // kpw
            kk = c % kpw
            return pltpu.make_async_copy(
                obuf.at[ob],
                out_hbm.at[h, :, pl.ds(wb0 + kk * BC, BC)],
                ssem.at[ob])

        def compute(c, gb, ob):
            h = c // kpw
            kk = c % kpw
            rows = rows_v.at[gb]

            def group(g, _):
                bb = g * L
                hoff = (idx_v[h, pl.ds(kk * BC + bb, L)] & jnp.int32(1)
                        ) * jnp.int32(D)
                rvec = iota + bb
                t = plsc.load_gather(rows, [rvec, hoff])
                acc = t * t
                for d in range(1, D):
                    t = plsc.load_gather(rows, [rvec, hoff + jnp.int32(d)])
                    acc = acc + t * t
                bits = plsc.bitcast(acc, jnp.int32)
                y = plsc.bitcast(jnp.int32(0x5F3759DF) - (bits >> 1),
                                 jnp.float32)
                ha = acc * jnp.float32(0.5)
                y = y * (jnp.float32(1.5) - ha * y * y)
                y = y * (jnp.float32(1.5) - ha * y * y)
                for d in range(D):
                    t = plsc.load_gather(rows, [rvec, hoff + jnp.int32(d)])
                    obuf[ob, d, pl.ds(bb, L)] = t * y
                return ()

            lax.fori_loop(0, BC // L, group, ())

        prep_gather(0, 0)
        prep_gather(1, 1)

        def outer_body(o, _):
            for u in range(NG):
                c = o * NG + u
                gb = u
                ob = u % NO

                @pl.when(c + 2 < n_chunks)
                def _():
                    prep_gather(c + 2, (u + 2) % NG)

                drain_gather(gb)

                @pl.when(c >= NO)
                def _():
                    store_slice(c - NO, ob).wait()

                compute(c, gb, ob)
                store_slice(c, ob).start()
            return ()

        lax.fori_loop(0, n_chunks // NG, outer_body, (), unroll=False)
        store_slice(n_chunks - 2, (n_chunks - 2) % NO).wait()
        store_slice(n_chunks - 1, (n_chunks - 1) % NO).wait()

    return k(table2, idx_t)


def kernel(x, table):
    b, h = x.shape
    v, d = table.shape
    idx_t = x.T.astype(jnp.int32)
    table2 = table.reshape(v // 2, W)
    out = _sc_embed_norm(table2, idx_t, batch=b, hist=h)
    return out.transpose(2, 0, 1)


# speedup vs baseline: 2.3450x; 2.3450x over previous
import functools

import jax
import jax.numpy as jnp
from jax import lax
from jax.experimental import pallas as pl
from jax.experimental.pallas import tpu as pltpu
from jax.experimental.pallas import tpu_sc as plsc

L = 16
D = 64
BC = 128
NG = 4
NO = 2


def _lane_shuffle(x, perm):
    dnums = lax.GatherDimensionNumbers(
        offset_dims=(), collapsed_slice_dims=(0,), start_index_map=(0,))
    return lax.gather(x, perm[:, None], dnums, slice_sizes=(1,),
                      mode=lax.GatherScatterMode.PROMISE_IN_BOUNDS)


def _eklundh16(m, iota):
    for s in (8, 4, 2, 1):
        perm = iota ^ s
        mask = (iota & s) == 0
        new = list(m)
        for a in range(L):
            if a & s:
                continue
            b = a ^ s
            new[a] = jnp.where(mask, m[a], _lane_shuffle(m[b], perm))
            new[b] = jnp.where(mask, _lane_shuffle(m[a], perm), m[b])
        m = new
    return m


def _sc_embed_norm(table, idx_t, *, batch, hist):
    info = plsc.get_sparse_core_info()
    nc, ns = info.num_cores, info.num_subcores
    nw = nc * ns
    b_per_w = batch // nw
    kpw = b_per_w // BC
    n_chunks = hist * kpw
    assert n_chunks % NG == 0

    mesh = plsc.VectorSubcoreMesh(core_axis_name="c", subcore_axis_name="s")

    @functools.partial(
        pl.kernel,
        out_type=jax.ShapeDtypeStruct((hist, D, batch), jnp.float32),
        mesh=mesh,
        scratch_types=[
            pltpu.VMEM((hist, b_per_w), jnp.int32),
            pltpu.VMEM((NG, BC, D), jnp.float32),
            pltpu.VMEM((NO, D, BC), jnp.float32),
            pltpu.SemaphoreType.DMA((NG,)),
            pltpu.SemaphoreType.DMA((NO,)),
        ],
        compiler_params=pltpu.CompilerParams(
            needs_layout_passes=False, use_tc_tiling_on_sc=False),
    )
    def k(table_hbm, idx_hbm, out_hbm, idx_v, rows_v, obuf, gsem, ssem):
        iota = lax.iota(jnp.int32, L)
        wid = lax.axis_index("s") * nc + lax.axis_index("c")
        wb0 = wid * b_per_w
        pltpu.sync_copy(idx_hbm.at[:, pl.ds(wb0, b_per_w)], idx_v)

        def gather(c, gb):
            h = c // kpw
            kk = c % kpw
            return pltpu.make_async_copy(
                table_hbm.at[idx_v.at[h, pl.ds(kk * BC, BC)]],
                rows_v.at[gb], gsem.at[gb])

        def store(c, ob):
            h = c // kpw
            kk = c % kpw
            return pltpu.make_async_copy(
                obuf.at[ob],
                out_hbm.at[h, :, pl.ds(wb0 + kk * BC, BC)],
                ssem.at[ob])

        def compute(gb, ob):
            rows = rows_v.at[gb]

            def group(g, _):
                bb = g * L
                ys = []
                for j in range(L):
                    v = [rows[bb + j, pl.ds(i * L, L)] for i in range(D // L)]
                    s = v[0] * v[0]
                    for i in range(1, D // L):
                        s = s + v[i] * v[i]
                    for sh in (8, 4, 2, 1):
                        s = s + _lane_shuffle(s, iota ^ sh)
                    bits = plsc.bitcast(s, jnp.int32)
                    y = plsc.bitcast(jnp.int32(0x5F3759DF) - (bits >> 1),
                                     jnp.float32)
                    hs = s * jnp.float32(0.5)
                    y = y * (jnp.float32(1.5) - hs * y * y)
                    y = y * (jnp.float32(1.5) - hs * y * y)
                    ys.append(y)
                for i in range(D // L):
                    m = [rows[bb + j, pl.ds(i * L, L)] * ys[j]
                         for j in range(L)]
                    t = _eklundh16(m, iota)
                    for j in range(L):
                        obuf[ob, i * L + j, pl.ds(bb, L)] = t[j]
                return ()

            lax.fori_loop(0, BC // L, group, ())

        gather(0, 0).start()
        gather(1, 1).start()

        def outer_body(o, _):
            for u in range(NG):
                c = o * NG + u
                gb = u
                ob = u % NO

                @pl.when(c + 2 < n_chunks)
                def _():
                    gather(c + 2, (u + 2) % NG).start()

                gather(c, gb).wait()

                @pl.when(c >= NO)
                def _():
                    store(c - NO, ob).wait()

                compute(gb, ob)
                store(c, ob).start()
            return ()

        lax.fori_loop(0, n_chunks // NG, outer_body, (), unroll=False)
        store(n_chunks - 2, (n_chunks - 2) % NO).wait()
        store(n_chunks - 1, (n_chunks - 1) % NO).wait()

    return k(table, idx_t)


def kernel(x, table):
    b, h = x.shape
    v, d = table.shape
    idx_t = x.T.astype(jnp.int32)
    out = _sc_embed_norm(table, idx_t, batch=b, hist=h)
    return out.transpose(2, 0, 1)


# speedup vs baseline: 2.3933x; 1.0206x over previous
import functools

import jax
import jax.numpy as jnp
from jax import lax
from jax.experimental import pallas as pl
from jax.experimental.pallas import tpu as pltpu
from jax.experimental.pallas import tpu_sc as plsc

L = 16
D = 64
W = 128
BC = 128
NG = 4
NO = 2


def _lane_shuffle(x, perm):
    dnums = lax.GatherDimensionNumbers(
        offset_dims=(), collapsed_slice_dims=(0,), start_index_map=(0,))
    return lax.gather(x, perm[:, None], dnums, slice_sizes=(1,),
                      mode=lax.GatherScatterMode.PROMISE_IN_BOUNDS)


def _eklundh16(m, iota):
    for s in (8, 4, 2, 1):
        perm = iota ^ s
        mask = (iota & s) == 0
        new = list(m)
        for a in range(L):
            if a & s:
                continue
            b = a ^ s
            new[a] = jnp.where(mask, m[a], _lane_shuffle(m[b], perm))
            new[b] = jnp.where(mask, _lane_shuffle(m[a], perm), m[b])
        m = new
    return m


def _sc_embed_norm(table2, idx_t, *, batch, hist):
    info = plsc.get_sparse_core_info()
    nc, ns = info.num_cores, info.num_subcores
    nw = nc * ns
    b_per_w = batch // nw
    kpw = b_per_w // BC
    n_chunks = hist * kpw
    assert n_chunks % NG == 0

    mesh = plsc.VectorSubcoreMesh(core_axis_name="c", subcore_axis_name="s")

    @functools.partial(
        pl.kernel,
        out_type=jax.ShapeDtypeStruct((hist, D, batch), jnp.float32),
        mesh=mesh,
        scratch_types=[
            pltpu.VMEM((hist, b_per_w), jnp.int32),
            pltpu.VMEM((NG, W), jnp.int32),
            pltpu.VMEM((NG, BC, W), jnp.float32),
            pltpu.VMEM((NO, D, BC), jnp.float32),
            pltpu.SemaphoreType.DMA((NG,)),
            pltpu.SemaphoreType.DMA((NO,)),
        ],
        compiler_params=pltpu.CompilerParams(
            needs_layout_passes=False, use_tc_tiling_on_sc=True),
    )
    def k(table_hbm, idx_hbm, out_hbm, idx_v, m_v, rows_v, obuf, gsem, ssem):
        iota = lax.iota(jnp.int32, L)
        zeros = iota & 0
        wid = lax.axis_index("s") * nc + lax.axis_index("c")
        wb0 = wid * b_per_w
        pltpu.sync_copy(idx_hbm.at[:, pl.ds(wb0, b_per_w)], idx_v)

        def prep_gather(c, gb):
            h = c // kpw
            kk = c % kpw
            for i in range(W // L):
                m_v[gb, pl.ds(i * L, L)] = (
                    idx_v[h, pl.ds(kk * BC + i * L, L)] >> jnp.int32(1))
            pltpu.async_copy(table_hbm.at[m_v.at[gb]], rows_v.at[gb],
                             gsem.at[gb])

        def drain_gather(gb):
            pltpu.make_async_copy(table_hbm.at[m_v.at[gb]], rows_v.at[gb],
                                  gsem.at[gb]).wait()

        def store(c, ob):
            h = c // kpw
            kk = c % kpw
            return pltpu.make_async_copy(
                obuf.at[ob],
                out_hbm.at[h, :, pl.ds(wb0 + kk * BC, BC)],
                ssem.at[ob])

        def compute(c, gb, ob):
            h = c // kpw
            kk = c % kpw
            rows = rows_v.at[gb]

            def group(g, _):
                bb = g * L
                hb = idx_v[h, pl.ds(kk * BC + bb, L)] & jnp.int32(1)
                for j in range(L):
                    hm = _lane_shuffle(hb, zeros + j) != 0
                    r = bb + j
                    v = [jnp.where(hm,
                                   rows[r, pl.ds(D + i * L, L)],
                                   rows[r, pl.ds(i * L, L)])
                         for i in range(D // L)]
                    s = v[0] * v[0]
                    for i in range(1, D // L):
                        s = s + v[i] * v[i]
                    for sh in (8, 4, 2, 1):
                        s = s + _lane_shuffle(s, iota ^ sh)
                    bits = plsc.bitcast(s, jnp.int32)
                    y = plsc.bitcast(jnp.int32(0x5F3759DF) - (bits >> 1),
                                     jnp.float32)
                    hs = s * jnp.float32(0.5)
                    y = y * (jnp.float32(1.5) - hs * y * y)
                    y = y * (jnp.float32(1.5) - hs * y * y)
                    for i in range(D // L):
                        rows[r, pl.ds(i * L, L)] = v[i] * y
                for i in range(D // L):
                    m = [rows[bb + j, pl.ds(i * L, L)] for j in range(L)]
                    t = _eklundh16(m, iota)
                    for j in range(L):
                        obuf[ob, i * L + j, pl.ds(bb, L)] = t[j]
                return ()

            lax.fori_loop(0, BC // L, group, ())

        prep_gather(0, 0)
        prep_gather(1, 1)

        def outer_body(o, _):
            for u in range(NG):
                c = o * NG + u
                gb = u
                ob = u % NO

                @pl.when(c + 2 < n_chunks)
                def _():
                    prep_gather(c + 2, (u + 2) % NG)

                drain_gather(gb)

                @pl.when(c >= NO)
                def _():
                    store(c - NO, ob).wait()

                compute(c, gb, ob)
                store(c, ob).start()
            return ()

        lax.fori_loop(0, n_chunks // NG, outer_body, (), unroll=False)
        store(n_chunks - 2, (n_chunks - 2) % NO).wait()
        store(n_chunks - 1, (n_chunks - 1) % NO).wait()

    return k(table2, idx_t)


def kernel(x, table):
    b, h = x.shape
    v, d = table.shape
    idx_t = x.T.astype(jnp.int32)
    table2 = table.reshape(v // 2, W)
    out = _sc_embed_norm(table2, idx_t, batch=b, hist=h)
    return out.transpose(2, 0, 1)


# speedup vs baseline: 2.7482x; 1.1483x over previous
import functools

import jax
import jax.numpy as jnp
from jax import lax
from jax.experimental import pallas as pl
from jax.experimental.pallas import tpu as pltpu
from jax.experimental.pallas import tpu_sc as plsc

L = 16
D = 64
W = 128
BC = 128
NG = 4
NO = 2


def _lane_shuffle(x, perm):
    dnums = lax.GatherDimensionNumbers(
        offset_dims=(), collapsed_slice_dims=(0,), start_index_map=(0,))
    return lax.gather(x, perm[:, None], dnums, slice_sizes=(1,),
                      mode=lax.GatherScatterMode.PROMISE_IN_BOUNDS)


def _eklundh16(m, iota):
    for s in (8, 4, 2, 1):
        perm = iota ^ s
        mask = (iota & s) == 0
        new = list(m)
        for a in range(L):
            if a & s:
                continue
            b = a ^ s
            new[a] = jnp.where(mask, m[a], _lane_shuffle(m[b], perm))
            new[b] = jnp.where(mask, _lane_shuffle(m[a], perm), m[b])
        m = new
    return m


def _sc_embed_norm(table2, idx_t, *, batch, hist):
    info = plsc.get_sparse_core_info()
    nc, ns = info.num_cores, info.num_subcores
    nw = nc * ns
    b_per_w = batch // nw
    kpw = b_per_w // BC
    n_chunks = hist * kpw
    assert n_chunks % NG == 0

    mesh = plsc.VectorSubcoreMesh(core_axis_name="c", subcore_axis_name="s")

    @functools.partial(
        pl.kernel,
        out_type=jax.ShapeDtypeStruct((hist, D, batch), jnp.float32),
        mesh=mesh,
        scratch_types=[
            pltpu.VMEM((hist, b_per_w), jnp.int32),
            pltpu.VMEM((NG, BC, W), jnp.float32),
            pltpu.VMEM((NO, D, BC), jnp.float32),
            pltpu.SemaphoreType.DMA((NG,)),
            pltpu.SemaphoreType.DMA((NO,)),
        ],
        compiler_params=pltpu.CompilerParams(
            needs_layout_passes=False, use_tc_tiling_on_sc=True),
    )
    def k(table_hbm, idx_hbm, out_hbm, idx_v, rows_v, obuf, gsem, ssem):
        iota = lax.iota(jnp.int32, L)
        wid = lax.axis_index("s") * nc + lax.axis_index("c")
        wb0 = wid * b_per_w
        pltpu.sync_copy(idx_hbm.at[:, pl.ds(wb0, b_per_w)], idx_v)

        def gather_ref(c, gb):
            h = c // kpw
            kk = c % kpw
            return pltpu.make_async_copy(
                table_hbm.at[idx_v.at[h, pl.ds(kk * BC, BC)]],
                rows_v.at[gb], gsem.at[gb])

        def store(c, ob):
            h = c // kpw
            kk = c % kpw
            return pltpu.make_async_copy(
                obuf.at[ob],
                out_hbm.at[h, :, pl.ds(wb0 + kk * BC, BC)],
                ssem.at[ob])

        def compute(gb, ob):
            rows = rows_v.at[gb]

            def group(g, _):
                bb = g * L
                for j in range(L):
                    r = bb + j
                    v = [rows[r, pl.ds(i * L, L)] for i in range(D // L)]
                    s = v[0] * v[0]
                    for i in range(1, D // L):
                        s = s + v[i] * v[i]
                    for sh in (8, 4, 2, 1):
                        s = s + _lane_shuffle(s, iota ^ sh)
                    bits = plsc.bitcast(s, jnp.int32)
                    y = plsc.bitcast(jnp.int32(0x5F3759DF) - (bits >> 1),
                                     jnp.float32)
                    hs = s * jnp.float32(0.5)
                    y = y * (jnp.float32(1.5) - hs * y * y)
                    y = y * (jnp.float32(1.5) - hs * y * y)
                    for i in range(D // L):
                        rows[r, pl.ds(i * L, L)] = v[i] * y
                for i in range(D // L):
                    m = [rows[bb + j, pl.ds(i * L, L)] for j in range(L)]
                    t = _eklundh16(m, iota)
                    for j in range(L):
                        obuf[ob, i * L + j, pl.ds(bb, L)] = t[j]
                return ()

            lax.fori_loop(0, BC // L, group, ())

        gather_ref(0, 0).start()
        gather_ref(1, 1).start()

        def outer_body(o, _):
            for u in range(NG):
                c = o * NG + u
                gb = u
                ob = u % NO

                @pl.when(c + 2 < n_chunks)
                def _():
                    gather_ref(c + 2, (u + 2) % NG).start()

                gather_ref(c, gb).wait()

                @pl.when(c >= NO)
                def _():
                    store(c - NO, ob).wait()

                compute(gb, ob)
                store(c, ob).start()
            return ()

        lax.fori_loop(0, n_chunks // NG, outer_body, (), unroll=False)
        store(n_chunks - 2, (n_chunks - 2) % NO).wait()
        store(n_chunks - 1, (n_chunks - 1) % NO).wait()

    return k(table2, idx_t)


def kernel(x, table):
    b, h = x.shape
    v, d = table.shape
    idx_t = x.T.astype(jnp.int32)
    table2 = jnp.pad(table, ((0, 0), (0, W - d)))
    out = _sc_embed_norm(table2, idx_t, batch=b, hist=h)
    return out.transpose(2, 0, 1)


# speedup vs baseline: 2.9911x; 1.0884x over previous
import functools

import jax
import jax.numpy as jnp
from jax import lax
from jax.experimental import pallas as pl
from jax.experimental.pallas import tpu as pltpu
from jax.experimental.pallas import tpu_sc as plsc

L = 16
D = 64
W = 128
BC = 128
NG = 4
NO = 2


def _lane_shuffle(x, perm):
    dnums = lax.GatherDimensionNumbers(
        offset_dims=(), collapsed_slice_dims=(0,), start_index_map=(0,))
    return lax.gather(x, perm[:, None], dnums, slice_sizes=(1,),
                      mode=lax.GatherScatterMode.PROMISE_IN_BOUNDS)


def _eklundh16(m, iota):
    for s in (8, 4, 2, 1):
        perm = iota ^ s
        mask = (iota & s) == 0
        new = list(m)
        for a in range(L):
            if a & s:
                continue
            b = a ^ s
            new[a] = jnp.where(mask, m[a], _lane_shuffle(m[b], perm))
            new[b] = jnp.where(mask, _lane_shuffle(m[a], perm), m[b])
        m = new
    return m


def _sc_embed_norm(table2, idx_t, *, batch, hist):
    info = plsc.get_sparse_core_info()
    nc, ns = info.num_cores, info.num_subcores
    nw = nc * ns
    b_per_w = batch // nw
    kpw = b_per_w // BC
    n_chunks = hist * kpw
    assert n_chunks % NG == 0

    mesh = plsc.VectorSubcoreMesh(core_axis_name="c", subcore_axis_name="s")

    @functools.partial(
        pl.kernel,
        out_type=jax.ShapeDtypeStruct((hist, D, batch), jnp.float32),
        mesh=mesh,
        scratch_types=[
            pltpu.VMEM((hist, b_per_w), jnp.int32),
            pltpu.VMEM((NG, BC, W), jnp.float32),
            pltpu.VMEM((NO, D, BC), jnp.float32),
            pltpu.SemaphoreType.DMA((NG,)),
            pltpu.SemaphoreType.DMA((NO,)),
        ],
        compiler_params=pltpu.CompilerParams(
            needs_layout_passes=False, use_tc_tiling_on_sc=True),
    )
    def k(table_hbm, idx_hbm, out_hbm, idx_v, rows_v, obuf, gsem, ssem):
        iota = lax.iota(jnp.int32, L)
        wid = lax.axis_index("s") * nc + lax.axis_index("c")
        wb0 = wid * b_per_w
        pltpu.sync_copy(idx_hbm.at[:, pl.ds(wb0, b_per_w)], idx_v)

        def gather_ref(c, gb):
            h = c // kpw
            kk = c % kpw
            return pltpu.make_async_copy(
                table_hbm.at[idx_v.at[h, pl.ds(kk * BC, BC)]],
                rows_v.at[gb], gsem.at[gb])

        def store(c, ob):
            h = c // kpw
            kk = c % kpw
            return pltpu.make_async_copy(
                obuf.at[ob],
                out_hbm.at[h, :, pl.ds(wb0 + kk * BC, BC)],
                ssem.at[ob])

        def compute(gb, ob):
            rows = rows_v.at[gb]

            def group(g, _):
                bb = g * L
                acc = None
                for i in range(D // L):
                    m = [rows[bb + j, pl.ds(i * L, L)] for j in range(L)]
                    t = _eklundh16(m, iota)
                    for j in range(L):
                        acc = t[j] * t[j] if acc is None else acc + t[j] * t[j]
                        obuf[ob, i * L + j, pl.ds(bb, L)] = t[j]
                bits = plsc.bitcast(acc, jnp.int32)
                y = plsc.bitcast(jnp.int32(0x5F3759DF) - (bits >> 1),
                                 jnp.float32)
                hs = acc * jnp.float32(0.5)
                y = y * (jnp.float32(1.5) - hs * y * y)
                y = y * (jnp.float32(1.5) - hs * y * y)
                for d in range(D):
                    obuf[ob, d, pl.ds(bb, L)] = obuf[ob, d, pl.ds(bb, L)] * y
                return ()

            lax.fori_loop(0, BC // L, group, ())

        gather_ref(0, 0).start()
        gather_ref(1, 1).start()

        def outer_body(o, _):
            for u in range(NG):
                c = o * NG + u
                gb = u
                ob = u % NO

                @pl.when(c + 2 < n_chunks)
                def _():
                    gather_ref(c + 2, (u + 2) % NG).start()

                gather_ref(c, gb).wait()

                @pl.when(c >= NO)
                def _():
                    store(c - NO, ob).wait()

                compute(gb, ob)
                store(c, ob).start()
            return ()

        lax.fori_loop(0, n_chunks // NG, outer_body, (), unroll=False)
        store(n_chunks - 2, (n_chunks - 2) % NO).wait()
        store(n_chunks - 1, (n_chunks - 1) % NO).wait()

    return k(table2, idx_t)


def kernel(x, table):
    b, h = x.shape
    v, d = table.shape
    idx_t = x.T.astype(jnp.int32)
    table2 = jnp.pad(table, ((0, 0), (0, W - d)))
    out = _sc_embed_norm(table2, idx_t, batch=b, hist=h)
    return out.transpose(2, 0, 1)
